# baseline (speedup 1.0000x reference)
import functools

import jax
import jax.numpy as jnp
from jax import lax
from jax.experimental import pallas as pl
from jax.experimental.pallas import tpu as pltpu
from jax.experimental.pallas import tpu_sc as plsc

N = 20000
D_VF = 2048
NUM_CLASSES = 91
M = 50000
THRESHOLD = 0.5


BLOCK_N = 800


def _head_body(x_ref, w_ref, b_ref, logits_ref, probs_ref):
    lg = jnp.dot(x_ref[...], w_ref[...], preferred_element_type=jnp.float32)
    lg = lg + b_ref[...]
    logits_ref[...] = lg
    probs_ref[...] = jax.nn.sigmoid(lg)


def _region_head(x, W, b2d):
    grid = (N // BLOCK_N,)
    return pl.pallas_call(
        _head_body,
        grid=grid,
        in_specs=[
            pl.BlockSpec((BLOCK_N, D_VF), lambda i: (i, 0)),
            pl.BlockSpec((D_VF, NUM_CLASSES), lambda i: (0, 0)),
            pl.BlockSpec((1, NUM_CLASSES), lambda i: (0, 0)),
        ],
        out_specs=[
            pl.BlockSpec((BLOCK_N, NUM_CLASSES), lambda i: (i, 0)),
            pl.BlockSpec((BLOCK_N, NUM_CLASSES), lambda i: (i, 0)),
        ],
        out_shape=[
            jax.ShapeDtypeStruct((N, NUM_CLASSES), jnp.float32),
            jax.ShapeDtypeStruct((N, NUM_CLASSES), jnp.float32),
        ],
    )(x, W, b2d)


NC = 2
NS = 16
NW = NC * NS
CHUNK = 128
N_CHUNKS = 13
BPW = CHUNK * N_CHUNKS
M_PAD = NW * BPW

_IOTA16 = None


def _gather_body(probs_hbm, boxes_hbm, labels_hbm, out_hbm,
                 i4_v, rows_v, labels_v, idx_v, gath_v, out_v, sem):
    wid = lax.axis_index("s") * NC + lax.axis_index("c")
    base = wid * BPW

    pltpu.sync_copy(labels_hbm.at[wid], labels_v)

    iota = lax.iota(jnp.int32, 16)
    for c in range(N_CHUNKS):
        for k in range(CHUNK // 16):
            j = c * (CHUNK // 16) + k
            i4_v[c, pl.ds(k * 16, 16)] = (base + j * 16 + iota) * 4

    row_copies = []
    for c in range(N_CHUNKS):
        row_copies.append(
            pltpu.async_copy(boxes_hbm.at[i4_v.at[c]], rows_v.at[c], sem))
    for cp in row_copies:
        cp.wait()

    for c in range(N_CHUNKS):
        for k in range(CHUNK // 16):
            rows = rows_v[c, pl.ds(k * 16, 16)]
            labs = labels_v[c, pl.ds(k * 16, 16)]
            idx_v[c, pl.ds(k * 16, 16)] = rows * NUM_CLASSES + labs

    copies = []
    for c in range(N_CHUNKS):
        copies.append(
            pltpu.async_copy(probs_hbm.at[idx_v.at[c]], gath_v.at[c], sem))
    for cp in copies:
        cp.wait()

    one = jnp.ones((16,), jnp.int32)
    zero = jnp.zeros((16,), jnp.int32)
    thr = jnp.full((16,), THRESHOLD, jnp.float32)
    for c in range(N_CHUNKS):
        for k in range(CHUNK // 16):
            g = gath_v[c, pl.ds(k * 16, 16)]
            out_v[c, pl.ds(k * 16, 16)] = jnp.where(g > thr, one, zero)

    pltpu.sync_copy(out_v, out_hbm.at[wid])


def _box_masks(probs_flat, boxes_pad, labels_pad):
    mesh = plsc.VectorSubcoreMesh(core_axis_name="c", subcore_axis_name="s")
    f = pl.kernel(
        _gather_body,
        out_type=jax.ShapeDtypeStruct((NW, N_CHUNKS, CHUNK), jnp.int32),
        mesh=mesh,
        scratch_types=[
            pltpu.VMEM((N_CHUNKS, CHUNK), jnp.int32),
            pltpu.VMEM((N_CHUNKS, CHUNK), jnp.int32),
            pltpu.VMEM((N_CHUNKS, CHUNK), jnp.int32),
            pltpu.VMEM((N_CHUNKS, CHUNK), jnp.int32),
            pltpu.VMEM((N_CHUNKS, CHUNK), jnp.float32),
            pltpu.VMEM((N_CHUNKS, CHUNK), jnp.int32),
            pltpu.SemaphoreType.DMA,
        ],
    )
    return f(probs_flat, boxes_pad, labels_pad)


def kernel(x, boxes, box_labels, W, b):
    logits, probs = _region_head(x, W, b.reshape(1, NUM_CLASSES))

    boxes_pad = jnp.zeros((M_PAD, 4), jnp.int32).at[:M].set(boxes)
    labels_pad = jnp.zeros((M_PAD,), jnp.int32).at[:M].set(box_labels)
    boxes_pad = boxes_pad.reshape(-1)
    labels_pad = labels_pad.reshape(NW, N_CHUNKS, CHUNK)

    mask_i32 = _box_masks(probs.reshape(-1), boxes_pad, labels_pad)
    box_masks = mask_i32.reshape(-1)[:M] != 0
    return (logits, probs, box_masks)


# speedup vs baseline: 1.0419x; 1.0419x over previous
import functools

import jax
import jax.numpy as jnp
from jax import lax
from jax.experimental import pallas as pl
from jax.experimental.pallas import tpu as pltpu
from jax.experimental.pallas import tpu_sc as plsc

N = 20000
D_VF = 2048
NUM_CLASSES = 91
M = 50000
THRESHOLD = 0.5
SEL_W = 128


BLOCK_N = 800


def _head_body(x_ref, w_ref, b_ref, logits_ref, probs_ref, sel_ref):
    lg = jnp.dot(x_ref[...], w_ref[...], preferred_element_type=jnp.float32)
    lg = lg + b_ref[...]
    logits_ref[...] = lg[:, :NUM_CLASSES]
    probs = jax.nn.sigmoid(lg)
    probs_ref[...] = probs[:, :NUM_CLASSES]
    sel_ref[...] = (probs > THRESHOLD).astype(jnp.int32)


def _region_head(x, W128, b128):
    grid = (N // BLOCK_N,)
    return pl.pallas_call(
        _head_body,
        grid=grid,
        in_specs=[
            pl.BlockSpec((BLOCK_N, D_VF), lambda i: (i, 0)),
            pl.BlockSpec((D_VF, SEL_W), lambda i: (0, 0)),
            pl.BlockSpec((1, SEL_W), lambda i: (0, 0)),
        ],
        out_specs=[
            pl.BlockSpec((BLOCK_N, NUM_CLASSES), lambda i: (i, 0)),
            pl.BlockSpec((BLOCK_N, NUM_CLASSES), lambda i: (i, 0)),
            pl.BlockSpec((BLOCK_N, SEL_W), lambda i: (i, 0)),
        ],
        out_shape=[
            jax.ShapeDtypeStruct((N, NUM_CLASSES), jnp.float32),
            jax.ShapeDtypeStruct((N, NUM_CLASSES), jnp.float32),
            jax.ShapeDtypeStruct((N, SEL_W), jnp.int32),
        ],
    )(x, W128, b128)


NC = 2
NS = 16
NW = NC * NS
CHUNK = 128
N_CHUNKS = 13
BPW = CHUNK * N_CHUNKS
M_PAD = NW * BPW


def _gather_body(sel_hbm, boxes_hbm, labels_hbm, out_hbm,
                 i4_v, rows_v, labels_v, idx_v, sem):
    wid = lax.axis_index("s") * NC + lax.axis_index("c")
    base = wid * BPW

    lab_cp = pltpu.async_copy(labels_hbm.at[wid], labels_v, sem)

    iota = lax.iota(jnp.int32, 16)
    for c in range(N_CHUNKS):
        for k in range(CHUNK // 16):
            j = c * (CHUNK // 16) + k
            i4_v[c, pl.ds(k * 16, 16)] = (base + j * 16 + iota) * 4

    row_copies = []
    for c in range(N_CHUNKS):
        row_copies.append(
            pltpu.async_copy(boxes_hbm.at[i4_v.at[c]], rows_v.at[c], sem))
    lab_cp.wait()
    for cp in row_copies:
        cp.wait()

    for c in range(N_CHUNKS):
        for k in range(CHUNK // 16):
            rows = rows_v[c, pl.ds(k * 16, 16)]
            labs = labels_v[c, pl.ds(k * 16, 16)]
            idx_v[c, pl.ds(k * 16, 16)] = rows * SEL_W + labs

    copies = []
    for c in range(N_CHUNKS):
        copies.append(
            pltpu.async_copy(sel_hbm.at[idx_v.at[c]], rows_v.at[c], sem))
    for cp in copies:
        cp.wait()

    pltpu.sync_copy(rows_v, out_hbm.at[wid])


def _box_masks(sel_flat, boxes_pad, labels_pad):
    mesh = plsc.VectorSubcoreMesh(core_axis_name="c", subcore_axis_name="s")
    f = pl.kernel(
        _gather_body,
        out_type=jax.ShapeDtypeStruct((NW, N_CHUNKS, CHUNK), jnp.int32),
        mesh=mesh,
        scratch_types=[
            pltpu.VMEM((N_CHUNKS, CHUNK), jnp.int32),
            pltpu.VMEM((N_CHUNKS, CHUNK), jnp.int32),
            pltpu.VMEM((N_CHUNKS, CHUNK), jnp.int32),
            pltpu.VMEM((N_CHUNKS, CHUNK), jnp.int32),
            pltpu.SemaphoreType.DMA,
        ],
    )
    return f(sel_flat, boxes_pad, labels_pad)


def kernel(x, boxes, box_labels, W, b):
    W128 = jnp.zeros((D_VF, SEL_W), jnp.float32).at[:, :NUM_CLASSES].set(W)
    b128 = jnp.zeros((1, SEL_W), jnp.float32).at[:, :NUM_CLASSES].set(
        b.reshape(1, NUM_CLASSES))
    logits, probs, sel = _region_head(x, W128, b128)

    boxes_pad = jnp.zeros((M_PAD, 4), jnp.int32).at[:M].set(boxes)
    labels_pad = jnp.zeros((M_PAD,), jnp.int32).at[:M].set(box_labels)
    boxes_pad = boxes_pad.reshape(-1)
    labels_pad = labels_pad.reshape(NW, N_CHUNKS, CHUNK)

    mask_i32 = _box_masks(sel.reshape(-1), boxes_pad, labels_pad)
    box_masks = mask_i32.reshape(-1)[:M] != 0
    return (logits, probs, box_masks)


# speedup vs baseline: 1.0482x; 1.0061x over previous
import functools

import jax
import jax.numpy as jnp
from jax import lax
from jax.experimental import pallas as pl
from jax.experimental.pallas import tpu as pltpu
from jax.experimental.pallas import tpu_sc as plsc

N = 20000
D_VF = 2048
NUM_CLASSES = 91
M = 50000
THRESHOLD = 0.5
SEL_W = 128


BLOCK_N = 2000


def _head_body(x_ref, w_ref, b_ref, logits_ref, probs_ref, sel_ref):
    lg = jnp.dot(x_ref[...], w_ref[...], preferred_element_type=jnp.float32)
    lg = lg + b_ref[...]
    logits_ref[...] = lg[:, :NUM_CLASSES]
    probs = jax.nn.sigmoid(lg)
    probs_ref[...] = probs[:, :NUM_CLASSES]
    sel_ref[...] = (probs > THRESHOLD).astype(jnp.int32)


def _region_head(x, W128, b128):
    grid = (N // BLOCK_N,)
    return pl.pallas_call(
        _head_body,
        grid=grid,
        in_specs=[
            pl.BlockSpec((BLOCK_N, D_VF), lambda i: (i, 0)),
            pl.BlockSpec((D_VF, SEL_W), lambda i: (0, 0)),
            pl.BlockSpec((1, SEL_W), lambda i: (0, 0)),
        ],
        out_specs=[
            pl.BlockSpec((BLOCK_N, NUM_CLASSES), lambda i: (i, 0)),
            pl.BlockSpec((BLOCK_N, NUM_CLASSES), lambda i: (i, 0)),
            pl.BlockSpec((BLOCK_N, SEL_W), lambda i: (i, 0)),
        ],
        out_shape=[
            jax.ShapeDtypeStruct((N, NUM_CLASSES), jnp.float32),
            jax.ShapeDtypeStruct((N, NUM_CLASSES), jnp.float32),
            jax.ShapeDtypeStruct((N, SEL_W), jnp.int32),
        ],
    )(x, W128, b128)


NC = 2
NS = 16
NW = NC * NS
CHUNK = 128
N_CHUNKS = 13
BPW = CHUNK * N_CHUNKS
M_PAD = NW * BPW


def _gather_body(sel_hbm, boxes_hbm, labels_hbm, out_hbm,
                 i4_v, rows_v, labels_v, idx_v, sem):
    wid = lax.axis_index("s") * NC + lax.axis_index("c")
    base = wid * BPW

    lab_cp = pltpu.async_copy(labels_hbm.at[wid], labels_v, sem)

    iota = lax.iota(jnp.int32, 16)
    for c in range(N_CHUNKS):
        for k in range(CHUNK // 16):
            j = c * (CHUNK // 16) + k
            i4_v[c, pl.ds(k * 16, 16)] = (base + j * 16 + iota) * 4

    row_copies = []
    for c in range(N_CHUNKS):
        row_copies.append(
            pltpu.async_copy(boxes_hbm.at[i4_v.at[c]], rows_v.at[c], sem))
    lab_cp.wait()
    for cp in row_copies:
        cp.wait()

    for c in range(N_CHUNKS):
        for k in range(CHUNK // 16):
            rows = rows_v[c, pl.ds(k * 16, 16)]
            labs = labels_v[c, pl.ds(k * 16, 16)]
            idx_v[c, pl.ds(k * 16, 16)] = rows * SEL_W + labs

    copies = []
    for c in range(N_CHUNKS):
        copies.append(
            pltpu.async_copy(sel_hbm.at[idx_v.at[c]], rows_v.at[c], sem))
    for cp in copies:
        cp.wait()

    pltpu.sync_copy(rows_v, out_hbm.at[wid])


def _box_masks(sel_flat, boxes_pad, labels_pad):
    mesh = plsc.VectorSubcoreMesh(core_axis_name="c", subcore_axis_name="s")
    f = pl.kernel(
        _gather_body,
        out_type=jax.ShapeDtypeStruct((NW, N_CHUNKS, CHUNK), jnp.int32),
        mesh=mesh,
        scratch_types=[
            pltpu.VMEM((N_CHUNKS, CHUNK), jnp.int32),
            pltpu.VMEM((N_CHUNKS, CHUNK), jnp.int32),
            pltpu.VMEM((N_CHUNKS, CHUNK), jnp.int32),
            pltpu.VMEM((N_CHUNKS, CHUNK), jnp.int32),
            pltpu.SemaphoreType.DMA,
        ],
    )
    return f(sel_flat, boxes_pad, labels_pad)


def kernel(x, boxes, box_labels, W, b):
    W128 = jnp.zeros((D_VF, SEL_W), jnp.float32).at[:, :NUM_CLASSES].set(W)
    b128 = jnp.zeros((1, SEL_W), jnp.float32).at[:, :NUM_CLASSES].set(
        b.reshape(1, NUM_CLASSES))
    logits, probs, sel = _region_head(x, W128, b128)

    boxes_pad = jnp.zeros((M_PAD, 4), jnp.int32).at[:M].set(boxes)
    labels_pad = jnp.zeros((M_PAD,), jnp.int32).at[:M].set(box_labels)
    boxes_pad = boxes_pad.reshape(-1)
    labels_pad = labels_pad.reshape(NW, N_CHUNKS, CHUNK)

    mask_i32 = _box_masks(sel.reshape(-1), boxes_pad, labels_pad)
    box_masks = mask_i32.reshape(-1)[:M] != 0
    return (logits, probs, box_masks)


# speedup vs baseline: 1.6023x; 1.5286x over previous
import functools

import jax
import jax.numpy as jnp
from jax import lax
from jax.experimental import pallas as pl
from jax.experimental.pallas import tpu as pltpu
from jax.experimental.pallas import tpu_sc as plsc

N = 20000
D_VF = 2048
NUM_CLASSES = 91
M = 50000
THRESHOLD = 0.5
SEL_W = 128


BLOCK_N = 1024


def _head_body(x_ref, w_ref, b_ref, logits_ref, probs_ref, sel_ref):
    lg = jnp.dot(x_ref[...], w_ref[...], preferred_element_type=jnp.float32)
    lg = lg + b_ref[...]
    probs = jax.nn.sigmoid(lg)
    logits_ref[...] = lg[:, :96].T
    probs_ref[...] = probs[:, :96].T
    sel_ref[...] = (probs > THRESHOLD).astype(jnp.int32)


def _region_head(x, W128, b128):
    grid = ((N + BLOCK_N - 1) // BLOCK_N,)
    return pl.pallas_call(
        _head_body,
        grid=grid,
        in_specs=[
            pl.BlockSpec((BLOCK_N, D_VF), lambda i: (i, 0)),
            pl.BlockSpec((D_VF, SEL_W), lambda i: (0, 0)),
            pl.BlockSpec((1, SEL_W), lambda i: (0, 0)),
        ],
        out_specs=[
            pl.BlockSpec((96, BLOCK_N), lambda i: (0, i)),
            pl.BlockSpec((96, BLOCK_N), lambda i: (0, i)),
            pl.BlockSpec((BLOCK_N, SEL_W), lambda i: (i, 0)),
        ],
        out_shape=[
            jax.ShapeDtypeStruct((96, N), jnp.float32),
            jax.ShapeDtypeStruct((96, N), jnp.float32),
            jax.ShapeDtypeStruct((N, SEL_W), jnp.int32),
        ],
    )(x, W128, b128)


NC = 2
NS = 16
NW = NC * NS
CHUNK = 128
N_CHUNKS = 13
BPW = CHUNK * N_CHUNKS
M_PAD = NW * BPW


def _gather_body(sel_hbm, rows_hbm, labels_hbm, out_hbm,
                 rows_v, labels_v, idx_v, sem):
    wid = lax.axis_index("s") * NC + lax.axis_index("c")

    row_cp = pltpu.async_copy(rows_hbm.at[wid], rows_v, sem)
    lab_cp = pltpu.async_copy(labels_hbm.at[wid], labels_v, sem)
    row_cp.wait()
    lab_cp.wait()

    for c in range(N_CHUNKS):
        for k in range(CHUNK // 16):
            rows = rows_v[c, pl.ds(k * 16, 16)]
            labs = labels_v[c, pl.ds(k * 16, 16)]
            idx_v[c, pl.ds(k * 16, 16)] = rows * SEL_W + labs

    copies = []
    for c in range(N_CHUNKS):
        copies.append(
            pltpu.async_copy(sel_hbm.at[idx_v.at[c]], rows_v.at[c], sem))
    for cp in copies:
        cp.wait()

    pltpu.sync_copy(rows_v, out_hbm.at[wid])


def _box_masks(sel_flat, rows_pad, labels_pad):
    mesh = plsc.VectorSubcoreMesh(core_axis_name="c", subcore_axis_name="s")
    f = pl.kernel(
        _gather_body,
        out_type=jax.ShapeDtypeStruct((NW, N_CHUNKS, CHUNK), jnp.int32),
        mesh=mesh,
        scratch_types=[
            pltpu.VMEM((N_CHUNKS, CHUNK), jnp.int32),
            pltpu.VMEM((N_CHUNKS, CHUNK), jnp.int32),
            pltpu.VMEM((N_CHUNKS, CHUNK), jnp.int32),
            pltpu.SemaphoreType.DMA,
        ],
    )
    return f(sel_flat, rows_pad, labels_pad)


def kernel(x, boxes, box_labels, W, b):
    W128 = jnp.zeros((D_VF, SEL_W), jnp.float32).at[:, :NUM_CLASSES].set(W)
    b128 = jnp.zeros((1, SEL_W), jnp.float32).at[:, :NUM_CLASSES].set(
        b.reshape(1, NUM_CLASSES))
    logits_t, probs_t, sel = _region_head(x, W128, b128)

    rows_pad = jnp.zeros((M_PAD,), jnp.int32).at[:M].set(boxes[:, 0])
    labels_pad = jnp.zeros((M_PAD,), jnp.int32).at[:M].set(box_labels)
    rows_pad = rows_pad.reshape(NW, N_CHUNKS, CHUNK)
    labels_pad = labels_pad.reshape(NW, N_CHUNKS, CHUNK)

    mask_i32 = _box_masks(sel.reshape(-1), rows_pad, labels_pad)
    box_masks = mask_i32.reshape(-1)[:M] != 0
    return (logits_t[:NUM_CLASSES].T, probs_t[:NUM_CLASSES].T, box_masks)
